# baseline (speedup 1.0000x reference)
import functools

import jax
import jax.numpy as jnp
from jax import lax
from jax.experimental import pallas as pl
from jax.experimental.pallas import tpu as pltpu
from jax.experimental.pallas import tpu_sc as plsc

_N = 10000
_D = 128
_NP = 10240
_BR = 1024
_GRID = _NP // _BR
_EPS = 1e-5

_EC = 128
_CH0 = 160
_CH1 = 0
_NB = 2
_NI = 2 * _NB
_EPAD = 16 * (_CH0 + _CH1) * _EC
_NROWS = _EPAD // _EC + _NI
_RPS = _NP // 16


def _sc_agg_body(h_hbm, src_hbm, dst_hbm, out_hbm, accum, sidx, didx, rows,
                 sem_g, sem_is, sem_id):
    c = lax.axis_index("c")
    s = lax.axis_index("s")

    pltpu.sync_copy(h_hbm.at[pl.ds(s * _RPS, _RPS)],
                    accum.at[pl.ds(s * _RPS, _RPS)])
    plsc.subcore_barrier()

    def _pipeline(ch, base):
        if ch == 0:
            return
        def _ld_idx(j, sl):
            return (pltpu.make_async_copy(src_hbm.at[base + j], sidx.at[sl],
                                          sem_is.at[sl]),
                    pltpu.make_async_copy(dst_hbm.at[base + j], didx.at[sl],
                                          sem_id.at[sl]))

        def _gather(sl, b):
            return pltpu.make_async_copy(h_hbm.at[sidx.at[sl]], rows.at[b],
                                         sem_g.at[b])

        for k in range(_NI):
            scp, dcp = _ld_idx(k, k)
            scp.start()
            dcp.start()

        for b in range(_NB):
            scp, _ = _ld_idx(b, b)
            scp.wait()
            _gather(b, b).start()

        def _visit(j, b, last):
            ib = j % _NI
            _gather(ib, b).wait()
            _, dcp = _ld_idx(j, ib)
            dcp.wait()
            pltpu.sync_copy(rows.at[b], accum.at[didx.at[ib]], add=True)
            scp, dcp = _ld_idx(j + _NI, ib)
            scp.start()
            dcp.start()
            if not last:
                jn = j + _NB
                scp2, _ = _ld_idx(jn, jn % _NI)
                scp2.wait()
                _gather(jn % _NI, b).start()

        def outer(grp, carry):
            for b in range(_NB):
                _visit(grp * _NB + b, b, False)
            return carry

        lax.fori_loop(0, ch // _NB - 1, outer, 0)
        for b in range(_NB):
            _visit((ch // _NB - 1) * _NB + b, b, True)

        for k in range(_NI):
            jo = ch + k
            scp, dcp = _ld_idx(jo, jo % _NI)
            scp.wait()
            dcp.wait()

    @pl.when(c == 0)
    def _():
        _pipeline(_CH0, s * _CH0)

    @pl.when(c != 0)
    def _():
        _pipeline(_CH1, 16 * _CH0 + s * _CH1)

    plsc.subcore_barrier()
    pltpu.sync_copy(accum.at[pl.ds(s * _RPS, _RPS)],
                    out_hbm.at[c, pl.ds(s * _RPS, _RPS)])


def _sc_agg(hp, src2, dst2):
    mesh = plsc.VectorSubcoreMesh(core_axis_name="c", subcore_axis_name="s")
    f = pl.kernel(
        _sc_agg_body, mesh=mesh,
        out_type=jax.ShapeDtypeStruct((2, _NP, _D), jnp.float32),
        scratch_types=[
            pltpu.VMEM_SHARED((_NP, _D), jnp.float32),
            pltpu.VMEM((_NI, _EC), jnp.int32),
            pltpu.VMEM((_NI, _EC), jnp.int32),
            pltpu.VMEM((_NB, _EC, _D), jnp.float32),
            pltpu.SemaphoreType.DMA((_NB,)),
            pltpu.SemaphoreType.DMA((_NI,)),
            pltpu.SemaphoreType.DMA((_NI,)),
        ],
    )
    return f(hp, src2, dst2)


def _mm_stats(X, w_ref, b_ref, y_ref, s_ref, ss_ref):
    Y = lax.dot_general(X, w_ref[...], (((1,), (0,)), ((), ())),
                        precision=lax.Precision.HIGHEST) + b_ref[...]
    y_ref[...] = Y
    i = pl.program_id(0)
    rid = lax.broadcasted_iota(jnp.int32, (_BR, 1), 0) + i * _BR
    Ym = jnp.where(rid < _N, Y, 0.0)

    @pl.when(i == 0)
    def _():
        s_ref[...] = jnp.zeros_like(s_ref)
        ss_ref[...] = jnp.zeros_like(ss_ref)

    s_ref[...] += jnp.sum(Ym, axis=0, keepdims=True)
    ss_ref[...] += jnp.sum(Ym * Ym, axis=0, keepdims=True)


def _k_agg_mm(p_ref, x_ref, w, b, y, s, ss):
    X = p_ref[0] + p_ref[1] - x_ref[...]
    _mm_stats(X, w, b, y, s, ss)


def _k_aff(yin, a, c, w, b, y, s, ss):
    X = jnp.maximum(yin[...] * a[...] + c[...], 0.0)
    _mm_stats(X, w, b, y, s, ss)


def _k_dual(p_ref, x_ref, q_ref, h_ref, wa, wb, b, y_ref, s_ref, ss_ref):
    X1 = p_ref[0] + p_ref[1] - x_ref[...]
    X2 = q_ref[0] + q_ref[1] - h_ref[...]
    Y = (lax.dot_general(X1, wa[...], (((1,), (0,)), ((), ())),
                         precision=lax.Precision.HIGHEST)
         + lax.dot_general(X2, wb[...], (((1,), (0,)), ((), ())),
                           precision=lax.Precision.HIGHEST) + b[...])
    y_ref[...] = Y
    i = pl.program_id(0)
    rid = lax.broadcasted_iota(jnp.int32, (_BR, 1), 0) + i * _BR
    Ym = jnp.where(rid < _N, Y, 0.0)

    @pl.when(i == 0)
    def _():
        s_ref[...] = jnp.zeros_like(s_ref)
        ss_ref[...] = jnp.zeros_like(ss_ref)

    s_ref[...] += jnp.sum(Ym, axis=0, keepdims=True)
    ss_ref[...] += jnp.sum(Ym * Ym, axis=0, keepdims=True)


def _k_out(yin, a, c, o):
    o[...] = jnp.maximum(yin[...] * a[...] + c[...], 0.0)


_ROWS = lambda: pl.BlockSpec((_BR, _D), lambda i: (i, 0))
_PAIR = lambda: pl.BlockSpec((2, _BR, _D), lambda i: (0, i, 0))
_WMAT = lambda: pl.BlockSpec((_D, _D), lambda i: (0, 0))
_VEC = lambda: pl.BlockSpec((1, _D), lambda i: (0, 0))

_MM_OUT = lambda: (
    [jax.ShapeDtypeStruct((_NP, _D), jnp.float32),
     jax.ShapeDtypeStruct((1, _D), jnp.float32),
     jax.ShapeDtypeStruct((1, _D), jnp.float32)],
    [_ROWS(), _VEC(), _VEC()],
)


def _call_agg_mm(P, x, w, b):
    out_shape, out_specs = _MM_OUT()
    return pl.pallas_call(
        _k_agg_mm, grid=(_GRID,),
        in_specs=[_PAIR(), _ROWS(), _WMAT(), _VEC()],
        out_specs=out_specs, out_shape=out_shape,
    )(P, x, w, b)


def _call_aff(yin, a, c, w, b):
    out_shape, out_specs = _MM_OUT()
    return pl.pallas_call(
        _k_aff, grid=(_GRID,),
        in_specs=[_ROWS(), _VEC(), _VEC(), _WMAT(), _VEC()],
        out_specs=out_specs, out_shape=out_shape,
    )(yin, a, c, w, b)


def _call_dual(P, x, Q, h, wa, wb, b):
    out_shape, out_specs = _MM_OUT()
    return pl.pallas_call(
        _k_dual, grid=(_GRID,),
        in_specs=[_PAIR(), _ROWS(), _PAIR(), _ROWS(), _WMAT(), _WMAT(),
                  _VEC()],
        out_specs=out_specs, out_shape=out_shape,
    )(P, x, Q, h, wa, wb, b)


def _call_out(yin, a, c):
    return pl.pallas_call(
        _k_out, grid=(_GRID,),
        in_specs=[_ROWS(), _VEC(), _VEC()],
        out_specs=_ROWS(),
        out_shape=jax.ShapeDtypeStruct((_NP, _D), jnp.float32),
    )(yin, a, c)


def _affine(s, ss, g, be):
    mean = s[0] / _N
    var = ss[0] / _N - mean * mean
    scale = g / jnp.sqrt(var + _EPS)
    shift = be - mean * scale
    return scale.reshape(1, _D), shift.reshape(1, _D)


def kernel(x, edge_index, params):
    p = params
    src = edge_index[0]
    dst = edge_index[1]

    xp = jnp.pad(x, ((0, _NP - _N), (0, 0)))
    epad = _NROWS * _EC - src.shape[0]
    src2 = jnp.concatenate([src, jnp.zeros((epad,), src.dtype)]
                           ).reshape(_NROWS, _EC)
    dst2 = jnp.concatenate([dst, jnp.full((epad,), _N, dst.dtype)]
                           ).reshape(_NROWS, _EC)

    b = lambda k: p[k].reshape(1, _D)

    P = _sc_agg(xp, src2, dst2)

    y1, s1, ss1 = _call_agg_mm(P, xp, p['W1'], b('b1'))
    sc1, sh1 = _affine(s1, ss1, p['g1'], p['be1'])
    y2, s2, ss2 = _call_aff(y1, sc1, sh1, p['W2'], b('b2'))
    sc2, sh2 = _affine(s2, ss2, p['g2'], p['be2'])
    h2 = _call_out(y2, sc2, sh2)

    Q = _sc_agg(h2, src2, dst2)

    y3, s3, ss3 = _call_dual(P, xp, Q, h2, p['W3'][:_D], p['W3'][_D:],
                             b('b3'))
    sc3, sh3 = _affine(s3, ss3, p['g3'], p['be3'])
    y4, s4, ss4 = _call_aff(y3, sc3, sh3, p['W4'], b('b4'))
    sc4, sh4 = _affine(s4, ss4, p['g4'], p['be4'])
    y5, s5, ss5 = _call_aff(y4, sc4, sh4, p['W5'], b('b5'))
    sc5, sh5 = _affine(s5, ss5, p['g5'], p['be5'])
    out = _call_out(y5, sc5, sh5)
    return out[:_N]


# speedup vs baseline: 3.2425x; 3.2425x over previous
import functools

import jax
import jax.numpy as jnp
from jax import lax
from jax.experimental import pallas as pl
from jax.experimental.pallas import tpu as pltpu
from jax.experimental.pallas import tpu_sc as plsc

_N = 10000
_D = 128
_NP = 10240
_BR = 1024
_GRID = _NP // _BR
_EPS = 1e-5

_EC = 128
_CH0 = 80
_CH1 = 80
_NB = 2
_NI = 2 * _NB
_EPAD = 16 * (_CH0 + _CH1) * _EC
_NROWS = _EPAD // _EC + _NI
_RPS = _NP // 16


def _sc_agg_body(h_hbm, src_hbm, dst_hbm, out_hbm, accum, sidx, didx, rows,
                 sem_g, sem_is, sem_id):
    c = lax.axis_index("c")
    s = lax.axis_index("s")

    pltpu.sync_copy(h_hbm.at[pl.ds(s * _RPS, _RPS)],
                    accum.at[pl.ds(s * _RPS, _RPS)])
    plsc.subcore_barrier()

    def _pipeline(ch, base):
        if ch == 0:
            return
        def _ld_idx(j, sl):
            return (pltpu.make_async_copy(src_hbm.at[base + j], sidx.at[sl],
                                          sem_is.at[sl]),
                    pltpu.make_async_copy(dst_hbm.at[base + j], didx.at[sl],
                                          sem_id.at[sl]))

        def _gather(sl, b):
            return pltpu.make_async_copy(h_hbm.at[sidx.at[sl]], rows.at[b],
                                         sem_g.at[b])

        for k in range(_NI):
            scp, dcp = _ld_idx(k, k)
            scp.start()
            dcp.start()

        for b in range(_NB):
            scp, _ = _ld_idx(b, b)
            scp.wait()
            _gather(b, b).start()

        def _visit(j, b, last):
            ib = j % _NI
            _gather(ib, b).wait()
            _, dcp = _ld_idx(j, ib)
            dcp.wait()
            pltpu.sync_copy(rows.at[b], accum.at[didx.at[ib]], add=True)
            scp, dcp = _ld_idx(j + _NI, ib)
            scp.start()
            dcp.start()
            if not last:
                jn = j + _NB
                scp2, _ = _ld_idx(jn, jn % _NI)
                scp2.wait()
                _gather(jn % _NI, b).start()

        def outer(grp, carry):
            for b in range(_NB):
                _visit(grp * _NB + b, b, False)
            return carry

        lax.fori_loop(0, ch // _NB - 1, outer, 0)
        for b in range(_NB):
            _visit((ch // _NB - 1) * _NB + b, b, True)

        for k in range(_NI):
            jo = ch + k
            scp, dcp = _ld_idx(jo, jo % _NI)
            scp.wait()
            dcp.wait()

    @pl.when(c == 0)
    def _():
        _pipeline(_CH0, s * _CH0)

    @pl.when(c != 0)
    def _():
        _pipeline(_CH1, 16 * _CH0 + s * _CH1)

    plsc.subcore_barrier()
    pltpu.sync_copy(accum.at[pl.ds(s * _RPS, _RPS)],
                    out_hbm.at[c, pl.ds(s * _RPS, _RPS)])


def _sc_agg(hp, src2, dst2):
    mesh = plsc.VectorSubcoreMesh(core_axis_name="c", subcore_axis_name="s")
    f = pl.kernel(
        _sc_agg_body, mesh=mesh,
        out_type=jax.ShapeDtypeStruct((2, _NP, _D), jnp.float32),
        scratch_types=[
            pltpu.VMEM_SHARED((_NP, _D), jnp.float32),
            pltpu.VMEM((_NI, _EC), jnp.int32),
            pltpu.VMEM((_NI, _EC), jnp.int32),
            pltpu.VMEM((_NB, _EC, _D), jnp.float32),
            pltpu.SemaphoreType.DMA((_NB,)),
            pltpu.SemaphoreType.DMA((_NI,)),
            pltpu.SemaphoreType.DMA((_NI,)),
        ],
    )
    return f(hp, src2, dst2)


def _mm_stats(X, w_ref, b_ref, y_ref, s_ref, ss_ref):
    Y = lax.dot_general(X, w_ref[...], (((1,), (0,)), ((), ())),
                        precision=lax.Precision.HIGHEST) + b_ref[...]
    y_ref[...] = Y
    i = pl.program_id(0)
    rid = lax.broadcasted_iota(jnp.int32, (_BR, 1), 0) + i * _BR
    Ym = jnp.where(rid < _N, Y, 0.0)

    @pl.when(i == 0)
    def _():
        s_ref[...] = jnp.zeros_like(s_ref)
        ss_ref[...] = jnp.zeros_like(ss_ref)

    s_ref[...] += jnp.sum(Ym, axis=0, keepdims=True)
    ss_ref[...] += jnp.sum(Ym * Ym, axis=0, keepdims=True)


def _k_agg_mm(p_ref, x_ref, w, b, y, s, ss):
    X = p_ref[0] + p_ref[1] - x_ref[...]
    _mm_stats(X, w, b, y, s, ss)


def _k_aff(yin, a, c, w, b, y, s, ss):
    X = jnp.maximum(yin[...] * a[...] + c[...], 0.0)
    _mm_stats(X, w, b, y, s, ss)


def _k_dual(p_ref, x_ref, q_ref, h_ref, wa, wb, b, y_ref, s_ref, ss_ref):
    X1 = p_ref[0] + p_ref[1] - x_ref[...]
    X2 = q_ref[0] + q_ref[1] - h_ref[...]
    Y = (lax.dot_general(X1, wa[...], (((1,), (0,)), ((), ())),
                         precision=lax.Precision.HIGHEST)
         + lax.dot_general(X2, wb[...], (((1,), (0,)), ((), ())),
                           precision=lax.Precision.HIGHEST) + b[...])
    y_ref[...] = Y
    i = pl.program_id(0)
    rid = lax.broadcasted_iota(jnp.int32, (_BR, 1), 0) + i * _BR
    Ym = jnp.where(rid < _N, Y, 0.0)

    @pl.when(i == 0)
    def _():
        s_ref[...] = jnp.zeros_like(s_ref)
        ss_ref[...] = jnp.zeros_like(ss_ref)

    s_ref[...] += jnp.sum(Ym, axis=0, keepdims=True)
    ss_ref[...] += jnp.sum(Ym * Ym, axis=0, keepdims=True)


def _k_out(yin, a, c, o):
    o[...] = jnp.maximum(yin[...] * a[...] + c[...], 0.0)


_ROWS = lambda: pl.BlockSpec((_BR, _D), lambda i: (i, 0))
_PAIR = lambda: pl.BlockSpec((2, _BR, _D), lambda i: (0, i, 0))
_WMAT = lambda: pl.BlockSpec((_D, _D), lambda i: (0, 0))
_VEC = lambda: pl.BlockSpec((1, _D), lambda i: (0, 0))

_MM_OUT = lambda: (
    [jax.ShapeDtypeStruct((_NP, _D), jnp.float32),
     jax.ShapeDtypeStruct((1, _D), jnp.float32),
     jax.ShapeDtypeStruct((1, _D), jnp.float32)],
    [_ROWS(), _VEC(), _VEC()],
)


def _call_agg_mm(P, x, w, b):
    out_shape, out_specs = _MM_OUT()
    return pl.pallas_call(
        _k_agg_mm, grid=(_GRID,),
        in_specs=[_PAIR(), _ROWS(), _WMAT(), _VEC()],
        out_specs=out_specs, out_shape=out_shape,
    )(P, x, w, b)


def _call_aff(yin, a, c, w, b):
    out_shape, out_specs = _MM_OUT()
    return pl.pallas_call(
        _k_aff, grid=(_GRID,),
        in_specs=[_ROWS(), _VEC(), _VEC(), _WMAT(), _VEC()],
        out_specs=out_specs, out_shape=out_shape,
    )(yin, a, c, w, b)


def _call_dual(P, x, Q, h, wa, wb, b):
    out_shape, out_specs = _MM_OUT()
    return pl.pallas_call(
        _k_dual, grid=(_GRID,),
        in_specs=[_PAIR(), _ROWS(), _PAIR(), _ROWS(), _WMAT(), _WMAT(),
                  _VEC()],
        out_specs=out_specs, out_shape=out_shape,
    )(P, x, Q, h, wa, wb, b)


def _call_out(yin, a, c):
    return pl.pallas_call(
        _k_out, grid=(_GRID,),
        in_specs=[_ROWS(), _VEC(), _VEC()],
        out_specs=_ROWS(),
        out_shape=jax.ShapeDtypeStruct((_NP, _D), jnp.float32),
    )(yin, a, c)


def _affine(s, ss, g, be):
    mean = s[0] / _N
    var = ss[0] / _N - mean * mean
    scale = g / jnp.sqrt(var + _EPS)
    shift = be - mean * scale
    return scale.reshape(1, _D), shift.reshape(1, _D)


def kernel(x, edge_index, params):
    p = params
    src = edge_index[0]
    dst = edge_index[1]

    xp = jnp.pad(x, ((0, _NP - _N), (0, 0)))
    epad = _NROWS * _EC - src.shape[0]
    fill = jnp.arange(epad, dtype=src.dtype)
    src2 = jnp.concatenate([src, fill % _N]).reshape(_NROWS, _EC)
    dst2 = jnp.concatenate([dst, _N + fill % (_NP - _N)]
                           ).reshape(_NROWS, _EC)

    b = lambda k: p[k].reshape(1, _D)

    P = _sc_agg(xp, src2, dst2)

    y1, s1, ss1 = _call_agg_mm(P, xp, p['W1'], b('b1'))
    sc1, sh1 = _affine(s1, ss1, p['g1'], p['be1'])
    y2, s2, ss2 = _call_aff(y1, sc1, sh1, p['W2'], b('b2'))
    sc2, sh2 = _affine(s2, ss2, p['g2'], p['be2'])
    h2 = _call_out(y2, sc2, sh2)

    Q = _sc_agg(h2, src2, dst2)

    y3, s3, ss3 = _call_dual(P, xp, Q, h2, p['W3'][:_D], p['W3'][_D:],
                             b('b3'))
    sc3, sh3 = _affine(s3, ss3, p['g3'], p['be3'])
    y4, s4, ss4 = _call_aff(y3, sc3, sh3, p['W4'], b('b4'))
    sc4, sh4 = _affine(s4, ss4, p['g4'], p['be4'])
    y5, s5, ss5 = _call_aff(y4, sc4, sh4, p['W5'], b('b5'))
    sc5, sh5 = _affine(s5, ss5, p['g5'], p['be5'])
    out = _call_out(y5, sc5, sh5)
    return out[:_N]


# speedup vs baseline: 3.7165x; 1.1462x over previous
import functools

import jax
import jax.numpy as jnp
from jax import lax
from jax.experimental import pallas as pl
from jax.experimental.pallas import tpu as pltpu
from jax.experimental.pallas import tpu_sc as plsc

_N = 10000
_D = 128
_BR = 1024
_GRID = 10
_EPS = 1e-5

_E = 320000
_EC = 80
_CH = 125
_NB = 4
_NI = 2 * _NB
_EPW = _CH * _EC
_IPAD = _NI * _EC
_NA = 10240
_RPS = 640


def _sc_agg_body(h_hbm, src_hbm, dst_hbm, out_hbm, accum, sidx, didx, rows,
                 sem_g, sem_is, sem_id):
    c = lax.axis_index("c")
    s = lax.axis_index("s")
    wid = s * 2 + c
    base = wid * _EPW

    ioff = pl.multiple_of(jnp.minimum(s * _RPS, _N - _RPS), 8)
    pltpu.sync_copy(h_hbm.at[pl.ds(ioff, _RPS)], accum.at[pl.ds(ioff, _RPS)])
    plsc.subcore_barrier()

    def _ld_idx(j, sl):
        return (pltpu.make_async_copy(src_hbm.at[pl.ds(base + j * _EC, _EC)],
                                      sidx.at[sl], sem_is.at[sl]),
                pltpu.make_async_copy(dst_hbm.at[pl.ds(base + j * _EC, _EC)],
                                      didx.at[sl], sem_id.at[sl]))

    def _gather(sl, b):
        return pltpu.make_async_copy(h_hbm.at[sidx.at[sl]], rows.at[b],
                                     sem_g.at[b])

    for k in range(_NI):
        scp, dcp = _ld_idx(k, k)
        scp.start()
        dcp.start()

    for b in range(_NB):
        scp, _ = _ld_idx(b, b)
        scp.wait()
        _gather(b, b).start()

    def _visit(j, b, issue_next):
        ib = j % _NI
        _gather(ib, b).wait()
        _, dcp = _ld_idx(j, ib)
        dcp.wait()
        pltpu.sync_copy(rows.at[b], accum.at[didx.at[ib]], add=True)
        scp, dcp = _ld_idx(j + _NI, ib)
        scp.start()
        dcp.start()
        if issue_next:
            jn = j + _NB
            scp2, _ = _ld_idx(jn, jn % _NI)
            scp2.wait()
            _gather(jn % _NI, b).start()

    ntail = _NB + (_CH % _NB)
    ngrp = (_CH - ntail) // _NB

    def outer(grp, carry):
        for b in range(_NB):
            _visit(grp * _NB + b, b, True)
        return carry

    lax.fori_loop(0, ngrp, outer, 0)
    for j in range(_CH - ntail, _CH):
        _visit(j, j % _NB, j + _NB < _CH)

    for k in range(_NI):
        jo = _CH + k
        scp, dcp = _ld_idx(jo, jo % _NI)
        scp.wait()
        dcp.wait()

    plsc.subcore_barrier()
    pltpu.sync_copy(accum.at[pl.ds(s * _RPS, _RPS)],
                    out_hbm.at[c, pl.ds(s * _RPS, _RPS)])


def _sc_agg(h, srcp, dstp):
    mesh = plsc.VectorSubcoreMesh(core_axis_name="c", subcore_axis_name="s")
    f = pl.kernel(
        _sc_agg_body, mesh=mesh,
        out_type=jax.ShapeDtypeStruct((2, _NA, _D), jnp.float32),
        scratch_types=[
            pltpu.VMEM_SHARED((_NA, _D), jnp.float32),
            pltpu.VMEM((_NI, _EC), jnp.int32),
            pltpu.VMEM((_NI, _EC), jnp.int32),
            pltpu.VMEM((_NB, _EC, _D), jnp.float32),
            pltpu.SemaphoreType.DMA((_NB,)),
            pltpu.SemaphoreType.DMA((_NI,)),
            pltpu.SemaphoreType.DMA((_NI,)),
        ],
    )
    return f(h, srcp, dstp)


def _dot(a, w):
    return lax.dot_general(a, w, (((1,), (0,)), ((), ())),
                           precision=lax.Precision.HIGHEST)


def _acc_stats(Y, i, s_ref, ss_ref):
    rid = lax.broadcasted_iota(jnp.int32, (_BR, 1), 0) + i * _BR
    Ym = jnp.where(rid < _N, Y, 0.0)

    @pl.when(i == 0)
    def _():
        s_ref[...] = jnp.zeros_like(s_ref)
        ss_ref[...] = jnp.zeros_like(ss_ref)

    s_ref[...] += jnp.sum(Ym, axis=0, keepdims=True)
    ss_ref[...] += jnp.sum(Ym * Ym, axis=0, keepdims=True)


def _bn_affine(s_ref, ss_ref, g_ref, be_ref):
    mean = s_ref[...] / _N
    var = ss_ref[...] / _N - mean * mean
    scale = g_ref[...] / jnp.sqrt(var + _EPS)
    shift = be_ref[...] - mean * scale
    return scale, shift


def _k_agg_mm(p_ref, x_ref, w, b, y_ref, s_ref, ss_ref):
    X = p_ref[0] + p_ref[1] - x_ref[...]
    Y = _dot(X, w[...]) + b[...]
    y_ref[...] = Y
    _acc_stats(Y, pl.program_id(0), s_ref, ss_ref)


def _k_aff(yin, sp, ssp, g, be, w, b, y_ref, s_ref, ss_ref):
    sc, sh = _bn_affine(sp, ssp, g, be)
    X = jnp.maximum(yin[...] * sc + sh, 0.0)
    Y = _dot(X, w[...]) + b[...]
    y_ref[...] = Y
    _acc_stats(Y, pl.program_id(0), s_ref, ss_ref)


def _k_mm_part(p_ref, x_ref, w, b, y_ref):
    X = p_ref[0] + p_ref[1] - x_ref[...]
    y_ref[...] = _dot(X, w[...]) + b[...]


def _k_dual_fin(ya_ref, q_ref, h_ref, wb, y_ref, s_ref, ss_ref):
    X2 = q_ref[0] + q_ref[1] - h_ref[...]
    Y = ya_ref[...] + _dot(X2, wb[...])
    y_ref[...] = Y
    _acc_stats(Y, pl.program_id(0), s_ref, ss_ref)


def _k_out(yin, sp, ssp, g, be, o):
    sc, sh = _bn_affine(sp, ssp, g, be)
    o[...] = jnp.maximum(yin[...] * sc + sh, 0.0)


_ROWS = lambda: pl.BlockSpec((_BR, _D), lambda i: (i, 0))
_PAIR = lambda: pl.BlockSpec((2, _BR, _D), lambda i: (0, i, 0))
_WMAT = lambda: pl.BlockSpec((_D, _D), lambda i: (0, 0))
_VEC = lambda: pl.BlockSpec((1, _D), lambda i: (0, 0))

_MM_OUT = lambda: (
    [jax.ShapeDtypeStruct((_N, _D), jnp.float32),
     jax.ShapeDtypeStruct((1, _D), jnp.float32),
     jax.ShapeDtypeStruct((1, _D), jnp.float32)],
    [_ROWS(), _VEC(), _VEC()],
)


def _call_agg_mm(P, x, w, b):
    out_shape, out_specs = _MM_OUT()
    return pl.pallas_call(
        _k_agg_mm, grid=(_GRID,),
        in_specs=[_PAIR(), _ROWS(), _WMAT(), _VEC()],
        out_specs=out_specs, out_shape=out_shape,
    )(P, x, w, b)


def _call_aff(yin, sp, ssp, g, be, w, b):
    out_shape, out_specs = _MM_OUT()
    return pl.pallas_call(
        _k_aff, grid=(_GRID,),
        in_specs=[_ROWS(), _VEC(), _VEC(), _VEC(), _VEC(), _WMAT(), _VEC()],
        out_specs=out_specs, out_shape=out_shape,
    )(yin, sp, ssp, g, be, w, b)


def _call_mm_part(P, x, w, b):
    return pl.pallas_call(
        _k_mm_part, grid=(_GRID,),
        in_specs=[_PAIR(), _ROWS(), _WMAT(), _VEC()],
        out_specs=_ROWS(),
        out_shape=jax.ShapeDtypeStruct((_N, _D), jnp.float32),
    )(P, x, w, b)


def _call_dual_fin(ya, Q, h, wb):
    out_shape, out_specs = _MM_OUT()
    return pl.pallas_call(
        _k_dual_fin, grid=(_GRID,),
        in_specs=[_ROWS(), _PAIR(), _ROWS(), _WMAT()],
        out_specs=out_specs, out_shape=out_shape,
    )(ya, Q, h, wb)


def _call_out(yin, sp, ssp, g, be):
    return pl.pallas_call(
        _k_out, grid=(_GRID,),
        in_specs=[_ROWS(), _VEC(), _VEC(), _VEC(), _VEC()],
        out_specs=_ROWS(),
        out_shape=jax.ShapeDtypeStruct((_N, _D), jnp.float32),
    )(yin, sp, ssp, g, be)


def kernel(x, edge_index, params):
    p = params
    ipad = jnp.zeros((_IPAD,), edge_index.dtype)
    srcp = jnp.concatenate([edge_index[0], ipad])
    dstp = jnp.concatenate([edge_index[1], ipad])

    b = lambda k: p[k].reshape(1, _D)
    v = lambda k: p[k].reshape(1, _D)

    P = _sc_agg(x, srcp, dstp)

    y1, s1, ss1 = _call_agg_mm(P, x, p['W1'], b('b1'))
    y2, s2, ss2 = _call_aff(y1, s1, ss1, v('g1'), v('be1'), p['W2'], b('b2'))
    h2 = _call_out(y2, s2, ss2, v('g2'), v('be2'))

    y3a = _call_mm_part(P, x, p['W3'][:_D], b('b3'))
    Q = _sc_agg(h2, srcp, dstp)
    y3, s3, ss3 = _call_dual_fin(y3a, Q, h2, p['W3'][_D:])

    y4, s4, ss4 = _call_aff(y3, s3, ss3, v('g3'), v('be3'), p['W4'], b('b4'))
    y5, s5, ss5 = _call_aff(y4, s4, ss4, v('g4'), v('be4'), p['W5'], b('b5'))
    return _call_out(y5, s5, ss5, v('g5'), v('be5'))


# speedup vs baseline: 3.8511x; 1.0362x over previous
import functools

import jax
import jax.numpy as jnp
from jax import lax
from jax.experimental import pallas as pl
from jax.experimental.pallas import tpu as pltpu
from jax.experimental.pallas import tpu_sc as plsc

_N = 10000
_D = 128
_BR = 1024
_GRID = 10
_EPS = 1e-5

_E = 320000
_EC = 80
_CH = 125
_NB = 4
_NI = 2 * _NB
_EPW = _CH * _EC
_IPAD = _NI * _EC
_NA = 10240
_RPS = 640


def _sc_agg_body(h_hbm, ei_hbm, out_hbm, accum, sidx, didx, rows,
                 sem_g, sem_is, sem_id):
    c = lax.axis_index("c")
    s = lax.axis_index("s")
    wid = s * 2 + c
    base = wid * _EPW

    ioff = pl.multiple_of(jnp.minimum(s * _RPS, _N - _RPS), 8)
    pltpu.sync_copy(h_hbm.at[pl.ds(ioff, _RPS)], accum.at[pl.ds(ioff, _RPS)])
    plsc.subcore_barrier()

    def _ld_idx(j, sl):
        off = pl.multiple_of(jnp.minimum(base + j * _EC, _E - _EC), 8)
        return (pltpu.make_async_copy(ei_hbm.at[pl.ds(off, _EC)],
                                      sidx.at[sl], sem_is.at[sl]),
                pltpu.make_async_copy(ei_hbm.at[pl.ds(_E + off, _EC)],
                                      didx.at[sl], sem_id.at[sl]))

    def _gather(sl, b):
        return pltpu.make_async_copy(h_hbm.at[sidx.at[sl]], rows.at[b],
                                     sem_g.at[b])

    for k in range(_NI):
        scp, dcp = _ld_idx(k, k)
        scp.start()
        dcp.start()

    for b in range(_NB):
        scp, _ = _ld_idx(b, b)
        scp.wait()
        _gather(b, b).start()

    def _visit(j, b, issue_next):
        ib = j % _NI
        _gather(ib, b).wait()
        _, dcp = _ld_idx(j, ib)
        dcp.wait()
        pltpu.sync_copy(rows.at[b], accum.at[didx.at[ib]], add=True)
        scp, dcp = _ld_idx(j + _NI, ib)
        scp.start()
        dcp.start()
        if issue_next:
            jn = j + _NB
            scp2, _ = _ld_idx(jn, jn % _NI)
            scp2.wait()
            _gather(jn % _NI, b).start()

    ntail = _NB + (_CH % _NB)
    ngrp = (_CH - ntail) // _NB

    def outer(grp, carry):
        for b in range(_NB):
            _visit(grp * _NB + b, b, True)
        return carry

    lax.fori_loop(0, ngrp, outer, 0)
    for j in range(_CH - ntail, _CH):
        _visit(j, j % _NB, j + _NB < _CH)

    for k in range(_NI):
        jo = _CH + k
        scp, dcp = _ld_idx(jo, jo % _NI)
        scp.wait()
        dcp.wait()

    plsc.subcore_barrier()
    pltpu.sync_copy(accum.at[pl.ds(s * _RPS, _RPS)],
                    out_hbm.at[c, pl.ds(s * _RPS, _RPS)])


def _sc_agg(h, ef):
    mesh = plsc.VectorSubcoreMesh(core_axis_name="c", subcore_axis_name="s")
    f = pl.kernel(
        _sc_agg_body, mesh=mesh,
        out_type=jax.ShapeDtypeStruct((2, _NA, _D), jnp.float32),
        scratch_types=[
            pltpu.VMEM_SHARED((_NA, _D), jnp.float32),
            pltpu.VMEM((_NI, _EC), jnp.int32),
            pltpu.VMEM((_NI, _EC), jnp.int32),
            pltpu.VMEM((_NB, _EC, _D), jnp.float32),
            pltpu.SemaphoreType.DMA((_NB,)),
            pltpu.SemaphoreType.DMA((_NI,)),
            pltpu.SemaphoreType.DMA((_NI,)),
        ],
    )
    return f(h, ef)


def _dot(a, w):
    return lax.dot_general(a, w, (((1,), (0,)), ((), ())),
                           precision=lax.Precision.HIGHEST)


def _acc_stats(Y, i, s_ref, ss_ref):
    rid = lax.broadcasted_iota(jnp.int32, (_BR, 1), 0) + i * _BR
    Ym = jnp.where(rid < _N, Y, 0.0)

    @pl.when(i == 0)
    def _():
        s_ref[...] = jnp.zeros_like(s_ref)
        ss_ref[...] = jnp.zeros_like(ss_ref)

    s_ref[...] += jnp.sum(Ym, axis=0, keepdims=True)
    ss_ref[...] += jnp.sum(Ym * Ym, axis=0, keepdims=True)


def _bn_affine(s_ref, ss_ref, g_ref, be_ref):
    mean = s_ref[...] / _N
    var = ss_ref[...] / _N - mean * mean
    scale = g_ref[...] / jnp.sqrt(var + _EPS)
    shift = be_ref[...] - mean * scale
    return scale, shift


def _k_agg_mm(p_ref, x_ref, w, b, y_ref, s_ref, ss_ref):
    X = p_ref[0] + p_ref[1] - x_ref[...]
    Y = _dot(X, w[...]) + b[...]
    y_ref[...] = Y
    _acc_stats(Y, pl.program_id(0), s_ref, ss_ref)


def _k_aff(yin, sp, ssp, g, be, w, b, y_ref, s_ref, ss_ref):
    sc, sh = _bn_affine(sp, ssp, g, be)
    X = jnp.maximum(yin[...] * sc + sh, 0.0)
    Y = _dot(X, w[...]) + b[...]
    y_ref[...] = Y
    _acc_stats(Y, pl.program_id(0), s_ref, ss_ref)


def _k_mm_part(p_ref, x_ref, w, b, y_ref):
    X = p_ref[0] + p_ref[1] - x_ref[...]
    y_ref[...] = _dot(X, w[...]) + b[...]


def _k_dual_fin(ya_ref, q_ref, h_ref, wb, y_ref, s_ref, ss_ref):
    X2 = q_ref[0] + q_ref[1] - h_ref[...]
    Y = ya_ref[...] + _dot(X2, wb[...])
    y_ref[...] = Y
    _acc_stats(Y, pl.program_id(0), s_ref, ss_ref)


def _k_out(yin, sp, ssp, g, be, o):
    sc, sh = _bn_affine(sp, ssp, g, be)
    o[...] = jnp.maximum(yin[...] * sc + sh, 0.0)


_ROWS = lambda: pl.BlockSpec((_BR, _D), lambda i: (i, 0))
_PAIR = lambda: pl.BlockSpec((2, _BR, _D), lambda i: (0, i, 0))
_WMAT = lambda: pl.BlockSpec((_D, _D), lambda i: (0, 0))
_VEC = lambda: pl.BlockSpec((1, _D), lambda i: (0, 0))

_MM_OUT = lambda: (
    [jax.ShapeDtypeStruct((_N, _D), jnp.float32),
     jax.ShapeDtypeStruct((1, _D), jnp.float32),
     jax.ShapeDtypeStruct((1, _D), jnp.float32)],
    [_ROWS(), _VEC(), _VEC()],
)


def _call_agg_mm(P, x, w, b):
    out_shape, out_specs = _MM_OUT()
    return pl.pallas_call(
        _k_agg_mm, grid=(_GRID,),
        in_specs=[_PAIR(), _ROWS(), _WMAT(), _VEC()],
        out_specs=out_specs, out_shape=out_shape,
    )(P, x, w, b)


def _call_aff(yin, sp, ssp, g, be, w, b):
    out_shape, out_specs = _MM_OUT()
    return pl.pallas_call(
        _k_aff, grid=(_GRID,),
        in_specs=[_ROWS(), _VEC(), _VEC(), _VEC(), _VEC(), _WMAT(), _VEC()],
        out_specs=out_specs, out_shape=out_shape,
    )(yin, sp, ssp, g, be, w, b)


def _call_mm_part(P, x, w, b):
    return pl.pallas_call(
        _k_mm_part, grid=(_GRID,),
        in_specs=[_PAIR(), _ROWS(), _WMAT(), _VEC()],
        out_specs=_ROWS(),
        out_shape=jax.ShapeDtypeStruct((_N, _D), jnp.float32),
    )(P, x, w, b)


def _call_dual_fin(ya, Q, h, wb):
    out_shape, out_specs = _MM_OUT()
    return pl.pallas_call(
        _k_dual_fin, grid=(_GRID,),
        in_specs=[_ROWS(), _PAIR(), _ROWS(), _WMAT()],
        out_specs=out_specs, out_shape=out_shape,
    )(ya, Q, h, wb)


def _call_out(yin, sp, ssp, g, be):
    return pl.pallas_call(
        _k_out, grid=(_GRID,),
        in_specs=[_ROWS(), _VEC(), _VEC(), _VEC(), _VEC()],
        out_specs=_ROWS(),
        out_shape=jax.ShapeDtypeStruct((_N, _D), jnp.float32),
    )(yin, sp, ssp, g, be)


def kernel(x, edge_index, params):
    p = params
    ef = edge_index.reshape(2 * _E)

    b = lambda k: p[k].reshape(1, _D)
    v = lambda k: p[k].reshape(1, _D)

    P = _sc_agg(x, ef)

    y1, s1, ss1 = _call_agg_mm(P, x, p['W1'], b('b1'))
    y2, s2, ss2 = _call_aff(y1, s1, ss1, v('g1'), v('be1'), p['W2'], b('b2'))
    h2 = _call_out(y2, s2, ss2, v('g2'), v('be2'))

    y3a = _call_mm_part(P, x, p['W3'][:_D], b('b3'))
    Q = _sc_agg(h2, ef)
    y3, s3, ss3 = _call_dual_fin(y3a, Q, h2, p['W3'][_D:])

    y4, s4, ss4 = _call_aff(y3, s3, ss3, v('g3'), v('be3'), p['W4'], b('b4'))
    y5, s5, ss5 = _call_aff(y4, s4, ss4, v('g4'), v('be4'), p['W5'], b('b5'))
    return _call_out(y5, s5, ss5, v('g5'), v('be5'))


# speedup vs baseline: 4.1594x; 1.0801x over previous
import functools

import jax
import jax.numpy as jnp
from jax import lax
from jax.experimental import pallas as pl
from jax.experimental.pallas import tpu as pltpu
from jax.experimental.pallas import tpu_sc as plsc

_N = 10000
_D = 128
_BR = 1024
_GRID = 10
_EPS = 1e-5

_E = 320000
_EC = 80
_CH = 125
_NB = 4
_NI = 2 * _NB
_EPW = _CH * _EC
_IPAD = _NI * _EC
_NA = 10240
_RPS = 640


def _sc_agg_body(h_hbm, ei_hbm, out_hbm, accum, sidx, didx, rows,
                 sem_g, sem_is, sem_id):
    c = lax.axis_index("c")
    s = lax.axis_index("s")
    wid = s * 2 + c
    base = wid * _EPW

    ioff = pl.multiple_of(jnp.minimum(s * _RPS, _N - _RPS), 8)
    pltpu.sync_copy(h_hbm.at[pl.ds(ioff, _RPS)], accum.at[pl.ds(ioff, _RPS)])
    plsc.subcore_barrier()

    def _ld_idx(j, sl):
        off = pl.multiple_of(jnp.minimum(base + j * _EC, _E - _EC), 8)
        return (pltpu.make_async_copy(ei_hbm.at[pl.ds(off, _EC)],
                                      sidx.at[sl], sem_is.at[sl]),
                pltpu.make_async_copy(ei_hbm.at[pl.ds(_E + off, _EC)],
                                      didx.at[sl], sem_id.at[sl]))

    def _gather(sl, b):
        return pltpu.make_async_copy(h_hbm.at[sidx.at[sl]], rows.at[b],
                                     sem_g.at[b])

    for k in range(_NI):
        scp, dcp = _ld_idx(k, k)
        scp.start()
        dcp.start()

    for b in range(_NB):
        scp, _ = _ld_idx(b, b)
        scp.wait()
        _gather(b, b).start()

    def _visit(j, b, issue_next):
        ib = j % _NI
        _gather(ib, b).wait()
        _, dcp = _ld_idx(j, ib)
        dcp.wait()
        pltpu.sync_copy(rows.at[b], accum.at[didx.at[ib]], add=True)
        scp, dcp = _ld_idx(j + _NI, ib)
        scp.start()
        dcp.start()
        if issue_next:
            jn = j + _NB
            scp2, _ = _ld_idx(jn, jn % _NI)
            scp2.wait()
            _gather(jn % _NI, b).start()

    ntail = _NB + (_CH % _NB)
    ngrp = (_CH - ntail) // _NB

    def outer(grp, carry):
        for b in range(_NB):
            _visit(grp * _NB + b, b, True)
        return carry

    lax.fori_loop(0, ngrp, outer, 0)
    for j in range(_CH - ntail, _CH):
        _visit(j, j % _NB, j + _NB < _CH)

    for k in range(_NI):
        jo = _CH + k
        scp, dcp = _ld_idx(jo, jo % _NI)
        scp.wait()
        dcp.wait()

    plsc.subcore_barrier()
    pltpu.sync_copy(accum.at[pl.ds(s * _RPS, _RPS)],
                    out_hbm.at[c, pl.ds(s * _RPS, _RPS)])


def _sc_agg(h, ef):
    mesh = plsc.VectorSubcoreMesh(core_axis_name="c", subcore_axis_name="s")
    f = pl.kernel(
        _sc_agg_body, mesh=mesh,
        out_type=jax.ShapeDtypeStruct((2, _NA, _D), jnp.float32),
        scratch_types=[
            pltpu.VMEM_SHARED((_NA, _D), jnp.float32),
            pltpu.VMEM((_NI, _EC), jnp.int32),
            pltpu.VMEM((_NI, _EC), jnp.int32),
            pltpu.VMEM((_NB, _EC, _D), jnp.float32),
            pltpu.SemaphoreType.DMA((_NB,)),
            pltpu.SemaphoreType.DMA((_NI,)),
            pltpu.SemaphoreType.DMA((_NI,)),
        ],
    )
    return f(h, ef)


def _dot(a, w):
    return lax.dot_general(a, w, (((1,), (0,)), ((), ())),
                           precision=lax.Precision.HIGHEST)


def _acc_stats(Y, k, first, s_ref, ss_ref):
    rid = lax.broadcasted_iota(jnp.int32, (_BR, 1), 0) + k * _BR
    Ym = jnp.where(rid < _N, Y, 0.0)

    @pl.when(first)
    def _():
        s_ref[...] = jnp.zeros_like(s_ref)
        ss_ref[...] = jnp.zeros_like(ss_ref)

    s_ref[...] += jnp.sum(Ym, axis=0, keepdims=True)
    ss_ref[...] += jnp.sum(Ym * Ym, axis=0, keepdims=True)


def _bn_affine(s_ref, ss_ref, g_ref, be_ref):
    mean = s_ref[...] / _N
    var = ss_ref[...] / _N - mean * mean
    scale = g_ref[...] / jnp.sqrt(var + _EPS)
    shift = be_ref[...] - mean * scale
    return scale, shift


def _k_mm_part(p_ref, x_ref, w, b, y_ref):
    X = p_ref[0] + p_ref[1] - x_ref[...]
    y_ref[...] = _dot(X, w[...]) + b[...]


def _k_mlp1(p_ref, x_ref, w1, b1, g1, be1, w2, b2, g2, be2, h2_ref,
            ybuf, sa, ssa, sb, ssb):
    i = pl.program_id(0)

    @pl.when(i < _GRID)
    def _():
        X = p_ref[0] + p_ref[1] - x_ref[...]
        Y = _dot(X, w1[...]) + b1[...]
        ybuf[pl.ds(i * _BR, _BR), :] = Y
        _acc_stats(Y, i, i == 0, sa, ssa)

    @pl.when((i >= _GRID) & (i < 2 * _GRID))
    def _():
        k = i - _GRID
        sc, sh = _bn_affine(sa, ssa, g1, be1)
        X = jnp.maximum(ybuf[pl.ds(k * _BR, _BR), :] * sc + sh, 0.0)
        Y = _dot(X, w2[...]) + b2[...]
        ybuf[pl.ds(k * _BR, _BR), :] = Y
        _acc_stats(Y, k, i == _GRID, sb, ssb)

    @pl.when(i >= 2 * _GRID)
    def _():
        k = i - 2 * _GRID
        sc, sh = _bn_affine(sb, ssb, g2, be2)
        h2_ref[...] = jnp.maximum(ybuf[pl.ds(k * _BR, _BR), :] * sc + sh, 0.0)


def _k_mlp2(ya_ref, q_ref, h_ref, w3b, g3, be3, w4, b4, g4, be4, w5, b5,
            g5, be5, o_ref, ybuf, sa, ssa, sb, ssb):
    i = pl.program_id(0)

    @pl.when(i < _GRID)
    def _():
        X2 = q_ref[0] + q_ref[1] - h_ref[...]
        Y = ya_ref[...] + _dot(X2, w3b[...])
        ybuf[pl.ds(i * _BR, _BR), :] = Y
        _acc_stats(Y, i, i == 0, sa, ssa)

    @pl.when((i >= _GRID) & (i < 2 * _GRID))
    def _():
        k = i - _GRID
        sc, sh = _bn_affine(sa, ssa, g3, be3)
        X = jnp.maximum(ybuf[pl.ds(k * _BR, _BR), :] * sc + sh, 0.0)
        Y = _dot(X, w4[...]) + b4[...]
        ybuf[pl.ds(k * _BR, _BR), :] = Y
        _acc_stats(Y, k, i == _GRID, sb, ssb)

    @pl.when((i >= 2 * _GRID) & (i < 3 * _GRID))
    def _():
        k = i - 2 * _GRID
        sc, sh = _bn_affine(sb, ssb, g4, be4)
        X = jnp.maximum(ybuf[pl.ds(k * _BR, _BR), :] * sc + sh, 0.0)
        Y = _dot(X, w5[...]) + b5[...]
        ybuf[pl.ds(k * _BR, _BR), :] = Y
        _acc_stats(Y, k, i == 2 * _GRID, sa, ssa)

    @pl.when(i >= 3 * _GRID)
    def _():
        k = i - 3 * _GRID
        sc, sh = _bn_affine(sa, ssa, g5, be5)
        o_ref[...] = jnp.maximum(ybuf[pl.ds(k * _BR, _BR), :] * sc + sh, 0.0)


_LAST = _GRID - 1
_ROWS = lambda: pl.BlockSpec((_BR, _D), lambda i: (i, 0))
_PAIR = lambda: pl.BlockSpec((2, _BR, _D), lambda i: (0, i, 0))
_WMAT = lambda: pl.BlockSpec((_D, _D), lambda i: (0, 0))
_VEC = lambda: pl.BlockSpec((1, _D), lambda i: (0, 0))
_ROWS0 = lambda: pl.BlockSpec((_BR, _D), lambda i: (jnp.minimum(i, _LAST), 0))
_PAIR0 = lambda: pl.BlockSpec((2, _BR, _D),
                              lambda i: (0, jnp.minimum(i, _LAST), 0))


def _out_phase(ph):
    return pl.BlockSpec((_BR, _D),
                        lambda i: (jnp.maximum(i - ph * _GRID, 0), 0))


_MLP_SCRATCH = lambda: [
    pltpu.VMEM((_GRID * _BR, _D), jnp.float32),
    pltpu.VMEM((1, _D), jnp.float32),
    pltpu.VMEM((1, _D), jnp.float32),
    pltpu.VMEM((1, _D), jnp.float32),
    pltpu.VMEM((1, _D), jnp.float32),
]


def _call_mm_part(P, x, w, b):
    return pl.pallas_call(
        _k_mm_part, grid=(_GRID,),
        in_specs=[_PAIR(), _ROWS(), _WMAT(), _VEC()],
        out_specs=_ROWS(),
        out_shape=jax.ShapeDtypeStruct((_N, _D), jnp.float32),
    )(P, x, w, b)


def _call_mlp1(P, x, w1, b1, g1, be1, w2, b2, g2, be2):
    return pl.pallas_call(
        _k_mlp1, grid=(3 * _GRID,),
        in_specs=[_PAIR0(), _ROWS0(), _WMAT(), _VEC(), _VEC(), _VEC(),
                  _WMAT(), _VEC(), _VEC(), _VEC()],
        out_specs=_out_phase(2),
        out_shape=jax.ShapeDtypeStruct((_N, _D), jnp.float32),
        scratch_shapes=_MLP_SCRATCH(),
    )(P, x, w1, b1, g1, be1, w2, b2, g2, be2)


def _call_mlp2(ya, Q, h2, w3b, g3, be3, w4, b4, g4, be4, w5, b5, g5, be5):
    return pl.pallas_call(
        _k_mlp2, grid=(4 * _GRID,),
        in_specs=[_ROWS0(), _PAIR0(), _ROWS0(), _WMAT(), _VEC(), _VEC(),
                  _WMAT(), _VEC(), _VEC(), _VEC(), _WMAT(), _VEC(), _VEC(),
                  _VEC()],
        out_specs=_out_phase(3),
        out_shape=jax.ShapeDtypeStruct((_N, _D), jnp.float32),
        scratch_shapes=_MLP_SCRATCH(),
    )(ya, Q, h2, w3b, g3, be3, w4, b4, g4, be4, w5, b5, g5, be5)


def kernel(x, edge_index, params):
    p = params
    ef = edge_index.reshape(2 * _E)

    b = lambda k: p[k].reshape(1, _D)
    v = lambda k: p[k].reshape(1, _D)

    P = _sc_agg(x, ef)

    h2 = _call_mlp1(P, x, p['W1'], b('b1'), v('g1'), v('be1'),
                    p['W2'], b('b2'), v('g2'), v('be2'))

    y3a = _call_mm_part(P, x, p['W3'][:_D], b('b3'))
    Q = _sc_agg(h2, ef)

    return _call_mlp2(y3a, Q, h2, p['W3'][_D:], v('g3'), v('be3'),
                      p['W4'], b('b4'), v('g4'), v('be4'),
                      p['W5'], b('b5'), v('g5'), v('be5'))


# speedup vs baseline: 4.1626x; 1.0008x over previous
import jax
import jax.numpy as jnp
from jax import lax
from jax.experimental import pallas as pl
from jax.experimental.pallas import tpu as pltpu
from jax.experimental.pallas import tpu_sc as plsc

_N = 10000
_D = 128
_BR = 1024
_GRID = 10
_EPS = 1e-5

_E = 320000
_EC = 80
_CH = 125
_NB = 4
_NI = 2 * _NB
_EPW = _CH * _EC
_NA = 10240
_RPS = 640


def _sc_agg_body(h_hbm, ei_hbm, out_hbm, accum, sidx, didx, rows,
                 sem_g, sem_is, sem_id):
    c = lax.axis_index("c")
    s = lax.axis_index("s")
    wid = s * 2 + c
    base = wid * _EPW

    ioff = pl.multiple_of(jnp.minimum(s * _RPS, _N - _RPS), 8)
    pltpu.sync_copy(h_hbm.at[pl.ds(ioff, _RPS)], accum.at[pl.ds(ioff, _RPS)])
    plsc.subcore_barrier()

    def _ld_idx(j, sl):
        off = pl.multiple_of(jnp.minimum(base + j * _EC, _E - _EC), 8)
        return (pltpu.make_async_copy(ei_hbm.at[pl.ds(off, _EC)],
                                      sidx.at[sl], sem_is.at[sl]),
                pltpu.make_async_copy(ei_hbm.at[pl.ds(_E + off, _EC)],
                                      didx.at[sl], sem_id.at[sl]))

    def _gather(sl, b):
        return pltpu.make_async_copy(h_hbm.at[sidx.at[sl]], rows.at[b],
                                     sem_g.at[b])

    for k in range(_NI):
        scp, dcp = _ld_idx(k, k)
        scp.start()
        dcp.start()

    for b in range(_NB):
        scp, _ = _ld_idx(b, b)
        scp.wait()
        _gather(b, b).start()

    def _visit(j, b, issue_next):
        ib = j % _NI
        _gather(ib, b).wait()
        _, dcp = _ld_idx(j, ib)
        dcp.wait()
        pltpu.sync_copy(rows.at[b], accum.at[didx.at[ib]], add=True)
        scp, dcp = _ld_idx(j + _NI, ib)
        scp.start()
        dcp.start()
        if issue_next:
            jn = j + _NB
            scp2, _ = _ld_idx(jn, jn % _NI)
            scp2.wait()
            _gather(jn % _NI, b).start()

    ntail = _NB + (_CH % _NB)
    ngrp = (_CH - ntail) // _NB

    def outer(grp, carry):
        for b in range(_NB):
            _visit(grp * _NB + b, b, True)
        return carry

    lax.fori_loop(0, ngrp, outer, 0)
    for j in range(_CH - ntail, _CH):
        _visit(j, j % _NB, j + _NB < _CH)

    for k in range(_NI):
        jo = _CH + k
        scp, dcp = _ld_idx(jo, jo % _NI)
        scp.wait()
        dcp.wait()

    plsc.subcore_barrier()
    pltpu.sync_copy(accum.at[pl.ds(s * _RPS, _RPS)],
                    out_hbm.at[c, pl.ds(s * _RPS, _RPS)])


def _sc_agg(h, ef):
    mesh = plsc.VectorSubcoreMesh(core_axis_name="c", subcore_axis_name="s")
    f = pl.kernel(
        _sc_agg_body, mesh=mesh,
        out_type=jax.ShapeDtypeStruct((2, _NA, _D), jnp.float32),
        scratch_types=[
            pltpu.VMEM_SHARED((_NA, _D), jnp.float32),
            pltpu.VMEM((_NI, _EC), jnp.int32),
            pltpu.VMEM((_NI, _EC), jnp.int32),
            pltpu.VMEM((_NB, _EC, _D), jnp.float32),
            pltpu.SemaphoreType.DMA((_NB,)),
            pltpu.SemaphoreType.DMA((_NI,)),
            pltpu.SemaphoreType.DMA((_NI,)),
        ],
    )
    return f(h, ef)


def _dot(a, w):
    return lax.dot_general(a, w, (((1,), (0,)), ((), ())),
                           precision=lax.Precision.HIGHEST)


def _acc_stats(Y, k, first, s_ref, ss_ref):
    rid = lax.broadcasted_iota(jnp.int32, (_BR, 1), 0) + k * _BR
    Ym = jnp.where(rid < _N, Y, 0.0)

    @pl.when(first)
    def _():
        s_ref[...] = jnp.zeros_like(s_ref)
        ss_ref[...] = jnp.zeros_like(ss_ref)

    s_ref[...] += jnp.sum(Ym, axis=0, keepdims=True)
    ss_ref[...] += jnp.sum(Ym * Ym, axis=0, keepdims=True)


def _bn_affine(s_ref, ss_ref, g_ref, be_ref):
    mean = s_ref[...] / _N
    var = ss_ref[...] / _N - mean * mean
    scale = g_ref[...] / jnp.sqrt(var + _EPS)
    shift = be_ref[...] - mean * scale
    return scale, shift


def _k_mm_part(p_ref, x_ref, w, b, y_ref):
    X = p_ref[0] + p_ref[1] - x_ref[...]
    y_ref[...] = _dot(X, w[...]) + b[...]


def _k_mlp1(p_ref, x_ref, w1, b1, g1, be1, w2, b2, g2, be2, h2_ref,
            ybuf, sa, ssa, sb, ssb):
    i = pl.program_id(0)

    @pl.when(i < _GRID)
    def _():
        X = p_ref[0] + p_ref[1] - x_ref[...]
        Y = _dot(X, w1[...]) + b1[...]
        ybuf[pl.ds(i * _BR, _BR), :] = Y
        _acc_stats(Y, i, i == 0, sa, ssa)

    @pl.when((i >= _GRID) & (i < 2 * _GRID))
    def _():
        k = i - _GRID
        sc, sh = _bn_affine(sa, ssa, g1, be1)
        X = jnp.maximum(ybuf[pl.ds(k * _BR, _BR), :] * sc + sh, 0.0)
        Y = _dot(X, w2[...]) + b2[...]
        ybuf[pl.ds(k * _BR, _BR), :] = Y
        _acc_stats(Y, k, i == _GRID, sb, ssb)

    @pl.when(i >= 2 * _GRID)
    def _():
        k = i - 2 * _GRID
        sc, sh = _bn_affine(sb, ssb, g2, be2)
        h2_ref[...] = jnp.maximum(ybuf[pl.ds(k * _BR, _BR), :] * sc + sh, 0.0)


def _k_mlp2(ya_ref, q_ref, h_ref, w3b, g3, be3, w4, b4, g4, be4, w5, b5,
            g5, be5, o_ref, ybuf, sa, ssa, sb, ssb):
    i = pl.program_id(0)

    @pl.when(i < _GRID)
    def _():
        X2 = q_ref[0] + q_ref[1] - h_ref[...]
        Y = ya_ref[...] + _dot(X2, w3b[...])
        ybuf[pl.ds(i * _BR, _BR), :] = Y
        _acc_stats(Y, i, i == 0, sa, ssa)

    @pl.when((i >= _GRID) & (i < 2 * _GRID))
    def _():
        k = i - _GRID
        sc, sh = _bn_affine(sa, ssa, g3, be3)
        X = jnp.maximum(ybuf[pl.ds(k * _BR, _BR), :] * sc + sh, 0.0)
        Y = _dot(X, w4[...]) + b4[...]
        ybuf[pl.ds(k * _BR, _BR), :] = Y
        _acc_stats(Y, k, i == _GRID, sb, ssb)

    @pl.when((i >= 2 * _GRID) & (i < 3 * _GRID))
    def _():
        k = i - 2 * _GRID
        sc, sh = _bn_affine(sb, ssb, g4, be4)
        X = jnp.maximum(ybuf[pl.ds(k * _BR, _BR), :] * sc + sh, 0.0)
        Y = _dot(X, w5[...]) + b5[...]
        ybuf[pl.ds(k * _BR, _BR), :] = Y
        _acc_stats(Y, k, i == 2 * _GRID, sa, ssa)

    @pl.when(i >= 3 * _GRID)
    def _():
        k = i - 3 * _GRID
        sc, sh = _bn_affine(sa, ssa, g5, be5)
        o_ref[...] = jnp.maximum(ybuf[pl.ds(k * _BR, _BR), :] * sc + sh, 0.0)


_LAST = _GRID - 1
_ROWS = lambda: pl.BlockSpec((_BR, _D), lambda i: (i, 0))
_PAIR = lambda: pl.BlockSpec((2, _BR, _D), lambda i: (0, i, 0))
_WMAT = lambda: pl.BlockSpec((_D, _D), lambda i: (0, 0))
_VEC = lambda: pl.BlockSpec((1, _D), lambda i: (0, 0))
_ROWS0 = lambda: pl.BlockSpec((_BR, _D), lambda i: (jnp.minimum(i, _LAST), 0))
_PAIR0 = lambda: pl.BlockSpec((2, _BR, _D),
                              lambda i: (0, jnp.minimum(i, _LAST), 0))


def _out_phase(ph):
    return pl.BlockSpec((_BR, _D),
                        lambda i: (jnp.maximum(i - ph * _GRID, 0), 0))


_MLP_SCRATCH = lambda: [
    pltpu.VMEM((_GRID * _BR, _D), jnp.float32),
    pltpu.VMEM((1, _D), jnp.float32),
    pltpu.VMEM((1, _D), jnp.float32),
    pltpu.VMEM((1, _D), jnp.float32),
    pltpu.VMEM((1, _D), jnp.float32),
]


def _call_mm_part(P, x, w, b):
    return pl.pallas_call(
        _k_mm_part, grid=(_GRID,),
        in_specs=[_PAIR(), _ROWS(), _WMAT(), _VEC()],
        out_specs=_ROWS(),
        out_shape=jax.ShapeDtypeStruct((_N, _D), jnp.float32),
    )(P, x, w, b)


def _call_mlp1(P, x, w1, b1, g1, be1, w2, b2, g2, be2):
    return pl.pallas_call(
        _k_mlp1, grid=(3 * _GRID,),
        in_specs=[_PAIR0(), _ROWS0(), _WMAT(), _VEC(), _VEC(), _VEC(),
                  _WMAT(), _VEC(), _VEC(), _VEC()],
        out_specs=_out_phase(2),
        out_shape=jax.ShapeDtypeStruct((_N, _D), jnp.float32),
        scratch_shapes=_MLP_SCRATCH(),
    )(P, x, w1, b1, g1, be1, w2, b2, g2, be2)


def _call_mlp2(ya, Q, h2, w3b, g3, be3, w4, b4, g4, be4, w5, b5, g5, be5):
    return pl.pallas_call(
        _k_mlp2, grid=(4 * _GRID,),
        in_specs=[_ROWS0(), _PAIR0(), _ROWS0(), _WMAT(), _VEC(), _VEC(),
                  _WMAT(), _VEC(), _VEC(), _VEC(), _WMAT(), _VEC(), _VEC(),
                  _VEC()],
        out_specs=_out_phase(3),
        out_shape=jax.ShapeDtypeStruct((_N, _D), jnp.float32),
        scratch_shapes=_MLP_SCRATCH(),
    )(ya, Q, h2, w3b, g3, be3, w4, b4, g4, be4, w5, b5, g5, be5)


def kernel(x, edge_index, params):
    p = params
    ef = edge_index.reshape(2 * _E)

    b = lambda k: p[k].reshape(1, _D)
    v = lambda k: p[k].reshape(1, _D)

    P = _sc_agg(x, ef)

    h2 = _call_mlp1(P, x, p['W1'], b('b1'), v('g1'), v('be1'),
                    p['W2'], b('b2'), v('g2'), v('be2'))

    y3a = _call_mm_part(P, x, p['W3'][:_D], b('b3'))
    Q = _sc_agg(h2, ef)

    return _call_mlp2(y3a, Q, h2, p['W3'][_D:], v('g3'), v('be3'),
                      p['W4'], b('b4'), v('g4'), v('be4'),
                      p['W5'], b('b5'), v('g5'), v('be5'))
